# Initial kernel scaffold; baseline (speedup 1.0000x reference)
import jax
import jax.numpy as jnp
from jax.experimental import pallas as pl


def kernel(x, A_indices, A_values, shape, W):
    raise NotImplementedError("write your pallas kernel here")


# speedup vs baseline: 4.0467x; 4.0467x over previous
import functools

import jax
import jax.numpy as jnp
from jax import lax
from jax.experimental import pallas as pl
from jax.experimental.pallas import tpu as pltpu
from jax.experimental.pallas import tpu_sc as plsc

N = 10000
NNZ = 160000
D_IN = 256
D_OUT = 256
DH = 128

NC = 2
NS = 16
EPT = NNZ // NS
CH = 80
NCH = EPT // CH
CPS = 25
NSC = NCH // CPS
N2 = 10112
RPT = N2 // NS

_mesh = plsc.VectorSubcoreMesh(
    core_axis_name="c", subcore_axis_name="s", num_cores=NC, num_subcores=NS
)


@functools.partial(
    pl.kernel,
    out_type=jax.ShapeDtypeStruct((NC, N2, DH), jnp.float32),
    mesh=_mesh,
    scratch_types=[
        pltpu.VMEM((CPS, CH), jnp.int32),
        pltpu.VMEM((CPS, CH), jnp.int32),
        pltpu.VMEM((CPS, CH), jnp.float32),
        pltpu.VMEM((CH, DH), jnp.float32),
        pltpu.VMEM_SHARED((N2, DH), jnp.float32),
        pltpu.SemaphoreType.DMA,
    ],
)
def _spmm_sc(xs, rows, cols, vals, zeros, out, row_v, col_v, val_v, gbuf, acc, sem):
    c = lax.axis_index("c")
    s = lax.axis_index("s")

    pltpu.sync_copy(zeros.at[pl.ds(s * RPT, RPT)], acc.at[pl.ds(s * RPT, RPT)])
    plsc.subcore_barrier()

    xh = xs.at[c]

    def super_body(sc, carry0):
        pltpu.sync_copy(rows.at[s].at[sc], row_v)
        pltpu.sync_copy(cols.at[s].at[sc], col_v)
        pltpu.sync_copy(vals.at[s].at[sc], val_v)

        def chunk_body(ci, carry):
            pltpu.async_copy(xh.at[col_v.at[ci]], gbuf, sem).wait()

            def group_body(g, carry2):
                vv = val_v[ci, pl.ds(g * 16, 16)]
                for e in range(16):
                    v = vv[e]
                    row = g * 16 + e
                    for j in range(DH // 16):
                        sl = pl.ds(j * 16, 16)
                        gbuf[row, sl] = gbuf[row, sl] * v
                return carry2

            lax.fori_loop(0, CH // 16, group_body, 0)

            pltpu.sync_copy(gbuf, acc.at[row_v.at[ci]], add=True)
            return carry

        lax.fori_loop(0, CPS, chunk_body, 0)
        return carry0

    lax.fori_loop(0, NSC, super_body, 0)
    plsc.subcore_barrier()

    osl = pl.ds(s * RPT, RPT)
    pltpu.sync_copy(acc.at[osl], out.at[c].at[osl])


_TM = 1000


def _linear_relu_body(hs_ref, wt_ref, o_ref):
    hl = hs_ref[0]
    hr = hs_ref[1]
    acc = jnp.dot(hl, wt_ref[:DH], preferred_element_type=jnp.float32)
    acc += jnp.dot(hr, wt_ref[DH:], preferred_element_type=jnp.float32)
    o_ref[...] = jnp.maximum(acc, 0.0)


_linear_relu = pl.pallas_call(
    _linear_relu_body,
    grid=(N // _TM,),
    in_specs=[
        pl.BlockSpec((NC, _TM, DH), lambda i: (0, i, 0)),
        pl.BlockSpec((D_IN, D_OUT), lambda i: (0, 0)),
    ],
    out_specs=pl.BlockSpec((_TM, D_OUT), lambda i: (i, 0)),
    out_shape=jax.ShapeDtypeStruct((N, D_OUT), jnp.float32),
)


def kernel(x, A_indices, A_values, shape, W):
    del shape
    rows = A_indices[0].astype(jnp.int32).reshape(NS, NSC, CPS, CH)
    cols = A_indices[1].astype(jnp.int32).reshape(NS, NSC, CPS, CH)
    vals = A_values.reshape(NS, NSC, CPS, CH)
    xs = jnp.stack([x[:, :DH], x[:, DH:]])
    zeros = jnp.zeros((N2, DH), jnp.float32)
    hs = _spmm_sc(xs, rows, cols, vals, zeros)[:, :N, :]
    return _linear_relu(hs, W.T.astype(jnp.float32))


# speedup vs baseline: 6.3907x; 1.5793x over previous
import functools

import jax
import jax.numpy as jnp
from jax import lax
from jax.experimental import pallas as pl
from jax.experimental.pallas import tpu as pltpu
from jax.experimental.pallas import tpu_sc as plsc

N = 10000
NNZ = 160000
D_IN = 256
D_OUT = 256
DH = 128

NC = 2
NS = 16
EPT = NNZ // NS
CH = 80
NCH = EPT // CH
CPS = 25
NSC = NCH // CPS
NPAIR = (CPS - 1) // 2 - 1
N2 = 10112
RPT = N2 // NS

_mesh = plsc.VectorSubcoreMesh(
    core_axis_name="c", subcore_axis_name="s", num_cores=NC, num_subcores=NS
)


@functools.partial(
    pl.kernel,
    out_type=jax.ShapeDtypeStruct((NC, N2, DH), jnp.float32),
    mesh=_mesh,
    scratch_types=[
        pltpu.VMEM((CPS, CH), jnp.int32),
        pltpu.VMEM((CPS, CH), jnp.int32),
        pltpu.VMEM((CPS, CH), jnp.float32),
        pltpu.VMEM((CH, DH), jnp.float32),
        pltpu.VMEM((CH, DH), jnp.float32),
        pltpu.VMEM_SHARED((N2, DH), jnp.float32),
        pltpu.SemaphoreType.DMA,
        pltpu.SemaphoreType.DMA,
        pltpu.SemaphoreType.DMA,
        pltpu.SemaphoreType.DMA,
    ],
)
def _spmm_sc(xs, rows, cols, vals, out,
             row_v, col_v, val_v, gba, gbb, acc, gsa, gsb, ssa, ssb):
    c = lax.axis_index("c")
    s = lax.axis_index("s")

    xh = xs.at[c]

    def zrow(i, carry):
        zv = jnp.zeros((16,), jnp.float32)
        for j in range(DH // 16):
            gba[i, pl.ds(j * 16, 16)] = zv
        return carry

    lax.fori_loop(0, CH, zrow, 0)
    base = s * RPT
    for i in range(7):
        pltpu.sync_copy(gba, acc.at[pl.ds(base + i * CH, CH)])
    pltpu.sync_copy(gba.at[pl.ds(0, RPT - 7 * CH)],
                    acc.at[pl.ds(base + 7 * CH, RPT - 7 * CH)])
    plsc.subcore_barrier()


    def g_issue(ci, buf, sem):
        pltpu.async_copy(xh.at[col_v.at[ci]], buf, sem)

    def g_wait(ci, buf, sem):
        pltpu.make_async_copy(xh.at[col_v.at[ci]], buf, sem).wait()

    def s_issue(ci, buf, sem):
        pltpu.async_copy(buf, acc.at[row_v.at[ci]], sem, add=True)

    def s_wait(ci, buf, sem):
        pltpu.make_async_copy(buf, acc.at[row_v.at[ci]], sem).wait()

    def scale(buf, ci):
        def group_body(g, carry):
            vv = val_v[ci, pl.ds(g * 16, 16)]
            for e in range(16):
                v = vv[e]
                row = g * 16 + e
                for j in range(DH // 16):
                    sl = pl.ds(j * 16, 16)
                    buf[row, sl] = buf[row, sl] * v
            return carry

        lax.fori_loop(0, CH // 16, group_body, 0)

    def super_body(sc, carry0):
        pltpu.sync_copy(rows.at[s].at[sc], row_v)
        pltpu.sync_copy(cols.at[s].at[sc], col_v)
        pltpu.sync_copy(vals.at[s].at[sc], val_v)

        g_issue(0, gba, gsa)
        g_issue(1, gbb, gsb)
        g_wait(0, gba, gsa)
        scale(gba, 0)
        s_issue(0, gba, ssa)

        def pair_body(k, carry):
            c1 = 2 * k + 1
            c2 = 2 * k + 2
            s_wait(c2 - 2, gba, ssa)
            g_issue(c2, gba, gsa)
            g_wait(c1, gbb, gsb)
            scale(gbb, c1)
            s_issue(c1, gbb, ssb)
            s_wait(c1, gbb, ssb)
            g_issue(c1 + 2, gbb, gsb)
            g_wait(c2, gba, gsa)
            scale(gba, c2)
            s_issue(c2, gba, ssa)
            return carry

        lax.fori_loop(0, NPAIR, pair_body, 0)

        s_wait(CPS - 3, gba, ssa)
        g_issue(CPS - 1, gba, gsa)
        g_wait(CPS - 2, gbb, gsb)
        scale(gbb, CPS - 2)
        s_issue(CPS - 2, gbb, ssb)
        g_wait(CPS - 1, gba, gsa)
        scale(gba, CPS - 1)
        s_issue(CPS - 1, gba, ssa)
        s_wait(CPS - 2, gbb, ssb)
        s_wait(CPS - 1, gba, ssa)
        return carry0

    lax.fori_loop(0, NSC, super_body, 0)
    plsc.subcore_barrier()

    osl = pl.ds(base, RPT)
    pltpu.sync_copy(acc.at[osl], out.at[c].at[osl])


_TM = 1000


def _linear_relu_body(hs_ref, wt_ref, o_ref):
    hl = hs_ref[0]
    hr = hs_ref[1]
    acc = jnp.dot(hl, wt_ref[:DH], preferred_element_type=jnp.float32)
    acc += jnp.dot(hr, wt_ref[DH:], preferred_element_type=jnp.float32)
    o_ref[...] = jnp.maximum(acc, 0.0)


_linear_relu = pl.pallas_call(
    _linear_relu_body,
    grid=(N // _TM,),
    in_specs=[
        pl.BlockSpec((NC, _TM, DH), lambda i: (0, i, 0)),
        pl.BlockSpec((D_IN, D_OUT), lambda i: (0, 0)),
    ],
    out_specs=pl.BlockSpec((_TM, D_OUT), lambda i: (i, 0)),
    out_shape=jax.ShapeDtypeStruct((N, D_OUT), jnp.float32),
)


def kernel(x, A_indices, A_values, shape, W):
    del shape
    rows = A_indices[0].astype(jnp.int32).reshape(NS, NSC, CPS, CH)
    cols = A_indices[1].astype(jnp.int32).reshape(NS, NSC, CPS, CH)
    vals = A_values.reshape(NS, NSC, CPS, CH)
    xs = jnp.stack([x[:, :DH], x[:, DH:]])
    hs = _spmm_sc(xs, rows, cols, vals)
    return _linear_relu(hs, W.T.astype(jnp.float32))
